# Initial kernel scaffold; baseline (speedup 1.0000x reference)
import functools

import jax
import jax.numpy as jnp
from jax import lax
from jax.experimental import pallas as pl
from jax.experimental.pallas import tpu as pltpu
from jax.experimental.pallas import tpu_sc as plsc

N_USERS = 10000
N_ENT = 10000
N_REL = 16
E_DIM = 128
N_NEI = 32
B = 4096
N_EDGES = 320000
N_NODES = N_USERS + N_ENT
N_LAYERS = 3

NC = 2
NS = 16
NW = NC * NS
HALF = N_NODES // NC
ACC_ROWS = HALF + 16
CHUNK = 128

SEG_CHUNKS = 160
SEG_PER_W = SEG_CHUNKS * CHUNK
SEG_PAD = SEG_PER_W * NS

DEG_CHUNKS = 157
DEG_PER_W = DEG_CHUNKS * CHUNK
DEG_PAD = DEG_PER_W * NW

B_PER_W = B // NW

_mesh = plsc.VectorSubcoreMesh(core_axis_name="c", subcore_axis_name="s")


def _zero_spmem(shared, zbuf, s):
    rows_total = shared.shape[0]
    width = shared.shape[1]
    zrows = zbuf.shape[0]
    per_s = rows_total // NS

    def zf(i, _):
        for j in range(width // 16):
            zbuf[i, pl.ds(j * 16, 16)] = jnp.zeros((16,), jnp.float32)
        return 0
    lax.fori_loop(0, zrows, zf, 0)

    base = s * per_s
    off = 0
    left = per_s
    while left > 0:
        n = min(zrows, left)
        pltpu.sync_copy(zbuf.at[pl.ds(0, n)], shared.at[pl.ds(base + off, n)])
        off += n
        left -= n


def _deg_body(idx_hbm, ones_hbm, out_hbm, idx_v, ones_v, zbuf, acc_sh, sem):
    c = lax.axis_index("c")
    s = lax.axis_index("s")
    w = c * NS + s
    _zero_spmem(acc_sh, zbuf, s)
    pltpu.sync_copy(ones_hbm, ones_v)
    pltpu.sync_copy(idx_hbm.at[w], idx_v)
    base = c * HALF

    def fix(g, _):
        for j in range(CHUNK // 16):
            d = idx_v[g, pl.ds(j * 16, 16)] - base
            ok = (d >= 0) & (d < HALF)
            idx_v[g, pl.ds(j * 16, 16)] = jnp.where(ok, d, HALF)
        return 0
    lax.fori_loop(0, DEG_CHUNKS, fix, 0)
    plsc.subcore_barrier()

    def chunk(g, _):
        pltpu.sync_copy(ones_v, acc_sh.at[idx_v.at[g]], add=True)
        return 0
    lax.fori_loop(0, DEG_CHUNKS, chunk, 0)
    plsc.subcore_barrier()

    per_s = HALF // NS
    row0 = s * per_s
    off = 0
    for n in (128, 128, 128, 128, 113):
        pltpu.sync_copy(acc_sh.at[pl.ds(row0 + off, n)], zbuf.at[pl.ds(0, n)])
        pltpu.sync_copy(zbuf.at[pl.ds(0, n)],
                        out_hbm.at[pl.ds(base + row0 + off, n)])
        off += n


_deg_call = functools.partial(
    pl.kernel,
    out_type=[jax.ShapeDtypeStruct((N_NODES, 16), jnp.float32)],
    mesh=_mesh,
    scratch_types=[
        pltpu.VMEM((DEG_CHUNKS, CHUNK), jnp.int32),
        pltpu.VMEM((CHUNK, 16), jnp.float32),
        pltpu.VMEM((128, 16), jnp.float32),
        pltpu.VMEM_SHARED((ACC_ROWS, 16), jnp.float32),
        pltpu.SemaphoreType.DMA,
    ],
)(_deg_body)


def _seg_body(table_hbm, src_hbm, dst_hbm, out_hbm,
              src_v, dst_v, rows0, rows1, acc_sh, sem0, sem1):
    c = lax.axis_index("c")
    s = lax.axis_index("s")
    _zero_spmem(acc_sh, rows0, s)
    pltpu.sync_copy(src_hbm.at[s], src_v)
    pltpu.sync_copy(dst_hbm.at[s], dst_v)
    base = c * HALF

    def fix(g, _):
        for j in range(CHUNK // 16):
            d = dst_v[g, pl.ds(j * 16, 16)] - base
            ok = (d >= 0) & (d < HALF)
            dst_v[g, pl.ds(j * 16, 16)] = jnp.where(ok, d, HALF)
        return 0
    lax.fori_loop(0, SEG_CHUNKS, fix, 0)
    plsc.subcore_barrier()

    def start(g, buf, sem):
        pltpu.async_copy(table_hbm.at[src_v.at[g]], buf, sem)

    def fin(g, buf, sem):
        pltpu.async_copy(table_hbm.at[src_v.at[g]], buf, sem).wait()
        pltpu.sync_copy(buf, acc_sh.at[dst_v.at[g]], add=True)

    start(0, rows0, sem0)
    start(1, rows1, sem1)

    def pair(p, _):
        g0 = p * 2
        fin(g0, rows0, sem0)
        start(g0 + 2, rows0, sem0)
        fin(g0 + 1, rows1, sem1)
        start(g0 + 3, rows1, sem1)
        return 0
    lax.fori_loop(0, SEG_CHUNKS // 2 - 1, pair, 0)
    fin(SEG_CHUNKS - 2, rows0, sem0)
    fin(SEG_CHUNKS - 1, rows1, sem1)
    plsc.subcore_barrier()

    per_s = HALF // NS
    row0 = s * per_s
    off = 0
    for n in (128, 128, 128, 128, 113):
        pltpu.sync_copy(acc_sh.at[pl.ds(row0 + off, n)], rows0.at[pl.ds(0, n)])
        pltpu.sync_copy(rows0.at[pl.ds(0, n)],
                        out_hbm.at[pl.ds(base + row0 + off, n)])
        off += n


_seg_call = functools.partial(
    pl.kernel,
    out_type=[jax.ShapeDtypeStruct((N_NODES, E_DIM), jnp.float32)],
    mesh=_mesh,
    scratch_types=[
        pltpu.VMEM((SEG_CHUNKS, CHUNK), jnp.int32),
        pltpu.VMEM((SEG_CHUNKS, CHUNK), jnp.int32),
        pltpu.VMEM((CHUNK, E_DIM), jnp.float32),
        pltpu.VMEM((CHUNK, E_DIM), jnp.float32),
        pltpu.VMEM_SHARED((ACC_ROWS, E_DIM), jnp.float32),
        pltpu.SemaphoreType.DMA,
        pltpu.SemaphoreType.DMA,
    ],
)(_seg_body)


def _gath_body(user_table, entity_table, adj_relation, user_ids, item_ids,
               u_out, it_out, nr_out, idx_v, rows_v, small_v, sem):
    c = lax.axis_index("c")
    s = lax.axis_index("s")
    w = c * NS + s
    b0 = w * B_PER_W
    pltpu.sync_copy(user_ids.at[pl.ds(b0, B_PER_W)], idx_v)
    pltpu.async_copy(user_table.at[idx_v], rows_v, sem).wait()
    pltpu.sync_copy(rows_v, u_out.at[pl.ds(b0, B_PER_W)])
    pltpu.sync_copy(item_ids.at[pl.ds(b0, B_PER_W)], idx_v)
    pltpu.async_copy(entity_table.at[idx_v], rows_v, sem).wait()
    pltpu.sync_copy(rows_v, it_out.at[pl.ds(b0, B_PER_W)])
    pltpu.async_copy(adj_relation.at[idx_v], small_v, sem).wait()
    pltpu.sync_copy(small_v, nr_out.at[pl.ds(b0, B_PER_W)])


_gath_call = functools.partial(
    pl.kernel,
    out_type=[
        jax.ShapeDtypeStruct((B, E_DIM), jnp.float32),
        jax.ShapeDtypeStruct((B, E_DIM), jnp.float32),
        jax.ShapeDtypeStruct((B, N_NEI), jnp.int32),
    ],
    mesh=_mesh,
    scratch_types=[
        pltpu.VMEM((B_PER_W,), jnp.int32),
        pltpu.VMEM((B_PER_W, E_DIM), jnp.float32),
        pltpu.VMEM((B_PER_W, N_NEI), jnp.int32),
        pltpu.SemaphoreType.DMA,
    ],
)(_gath_body)


def _agg_body(entity_table, adj_entity, item_ids, att_hbm, agg_out,
              idx_v, ne_v, att_v, agg_v, rows0, rows1, sem0, sem1):
    c = lax.axis_index("c")
    s = lax.axis_index("s")
    w = c * NS + s
    b0 = w * B_PER_W
    pltpu.sync_copy(item_ids.at[pl.ds(b0, B_PER_W)], idx_v)
    pltpu.async_copy(adj_entity.at[idx_v], ne_v, sem0).wait()
    pltpu.sync_copy(att_hbm.at[pl.ds(b0, B_PER_W)], att_v)

    def start(i, buf, sem):
        pltpu.async_copy(entity_table.at[ne_v.at[i]], buf, sem)

    def item(i, buf):
        accs = [jnp.zeros((16,), jnp.float32) for _ in range(E_DIM // 16)]
        for n in range(N_NEI):
            bn = plsc.load_gather(
                att_v, [jnp.full((16,), 1, jnp.int32) * i,
                        jnp.full((16,), n, jnp.int32)])
            for v in range(E_DIM // 16):
                accs[v] = accs[v] + bn * buf[n, pl.ds(v * 16, 16)]
        for v in range(E_DIM // 16):
            agg_v[i, pl.ds(v * 16, 16)] = accs[v]

    def fin(i, buf, sem):
        pltpu.async_copy(entity_table.at[ne_v.at[i]], buf, sem).wait()
        item(i, buf)

    start(0, rows0, sem0)
    start(1, rows1, sem1)

    def pair(p, _):
        i0 = p * 2
        fin(i0, rows0, sem0)
        start(i0 + 2, rows0, sem0)
        fin(i0 + 1, rows1, sem1)
        start(i0 + 3, rows1, sem1)
        return 0
    lax.fori_loop(0, B_PER_W // 2 - 1, pair, 0)
    fin(B_PER_W - 2, rows0, sem0)
    fin(B_PER_W - 1, rows1, sem1)
    pltpu.sync_copy(agg_v, agg_out.at[pl.ds(b0, B_PER_W)])


_agg_call = functools.partial(
    pl.kernel,
    out_type=[jax.ShapeDtypeStruct((B, E_DIM), jnp.float32)],
    mesh=_mesh,
    scratch_types=[
        pltpu.VMEM((B_PER_W,), jnp.int32),
        pltpu.VMEM((B_PER_W, N_NEI), jnp.int32),
        pltpu.VMEM((B_PER_W, N_NEI), jnp.float32),
        pltpu.VMEM((B_PER_W, E_DIM), jnp.float32),
        pltpu.VMEM((N_NEI, E_DIM), jnp.float32),
        pltpu.VMEM((N_NEI, E_DIM), jnp.float32),
        pltpu.SemaphoreType.DMA,
        pltpu.SemaphoreType.DMA,
    ],
)(_agg_body)


def _u1_body(acc_table, user_ids, u1_out, idx_v, rows_v, sem):
    c = lax.axis_index("c")
    s = lax.axis_index("s")
    w = c * NS + s
    b0 = w * B_PER_W
    pltpu.sync_copy(user_ids.at[pl.ds(b0, B_PER_W)], idx_v)
    pltpu.async_copy(acc_table.at[idx_v], rows_v, sem).wait()
    pltpu.sync_copy(rows_v, u1_out.at[pl.ds(b0, B_PER_W)])


_u1_call = functools.partial(
    pl.kernel,
    out_type=[jax.ShapeDtypeStruct((B, E_DIM), jnp.float32)],
    mesh=_mesh,
    scratch_types=[
        pltpu.VMEM((B_PER_W,), jnp.int32),
        pltpu.VMEM((B_PER_W, E_DIM), jnp.float32),
        pltpu.SemaphoreType.DMA,
    ],
)(_u1_body)


_NBLK = (N_NODES + 127) // 128


def _pre_body(e0_ref, deg_ref, egos_ref):
    rs = lax.rsqrt(jnp.maximum(deg_ref[...], 1.0))
    egos_ref[...] = e0_ref[...] * rs


def _tc_pre(E0, deg2d):
    return pl.pallas_call(
        _pre_body,
        grid=(_NBLK,),
        in_specs=[pl.BlockSpec((128, E_DIM), lambda i: (i, 0)),
                  pl.BlockSpec((128, 1), lambda i: (i, 0))],
        out_specs=pl.BlockSpec((128, E_DIM), lambda i: (i, 0)),
        out_shape=jax.ShapeDtypeStruct((N_NODES, E_DIM), jnp.float32),
    )(E0, deg2d)


def _layer_body(ego_ref, acc_ref, raw_ref, deg_ref, w1_ref, b1_ref, w2_ref,
                b2_ref, ego_o, acc_o, egos_o):
    rs = lax.rsqrt(jnp.maximum(deg_ref[...], 1.0))
    ego = ego_ref[...]
    side = raw_ref[...] * rs
    a = side + ego
    m = side * ego
    h = (jnp.dot(a, w1_ref[...], preferred_element_type=jnp.float32,
                 precision=lax.Precision.HIGHEST)
         + jnp.dot(m, w2_ref[...], preferred_element_type=jnp.float32,
                   precision=lax.Precision.HIGHEST)
         + b1_ref[...] + b2_ref[...])
    h = jnp.where(h >= 0, h, 0.2 * h)
    nrm = jnp.sqrt(jnp.sum(h * h, axis=1, keepdims=True))
    hn = h / (nrm + 1e-12)
    ego_o[...] = hn
    acc_o[...] = acc_ref[...] + hn
    egos_o[...] = hn * rs


def _tc_layer(ego, acc, raw, deg2d, w1, b1, w2, b2):
    return pl.pallas_call(
        _layer_body,
        grid=(_NBLK,),
        in_specs=[pl.BlockSpec((128, E_DIM), lambda i: (i, 0)),
                  pl.BlockSpec((128, E_DIM), lambda i: (i, 0)),
                  pl.BlockSpec((128, E_DIM), lambda i: (i, 0)),
                  pl.BlockSpec((128, 1), lambda i: (i, 0)),
                  pl.BlockSpec((E_DIM, E_DIM), lambda i: (0, 0)),
                  pl.BlockSpec((1, E_DIM), lambda i: (0, 0)),
                  pl.BlockSpec((E_DIM, E_DIM), lambda i: (0, 0)),
                  pl.BlockSpec((1, E_DIM), lambda i: (0, 0))],
        out_specs=[pl.BlockSpec((128, E_DIM), lambda i: (i, 0)),
                   pl.BlockSpec((128, E_DIM), lambda i: (i, 0)),
                   pl.BlockSpec((128, E_DIM), lambda i: (i, 0))],
        out_shape=[jax.ShapeDtypeStruct((N_NODES, E_DIM), jnp.float32),
                   jax.ShapeDtypeStruct((N_NODES, E_DIM), jnp.float32),
                   jax.ShapeDtypeStruct((N_NODES, E_DIM), jnp.float32)],
    )(ego, acc, raw, deg2d, w1, b1, w2, b2)


def _att_body(u_ref, rel_ref, nr_ref, att_ref):
    d = lax.dot_general(u_ref[...], rel_ref[...],
                        (((1,), (1,)), ((), ())),
                        preferred_element_type=jnp.float32,
                        precision=lax.Precision.HIGHEST)
    nr = nr_ref[...]
    logits = jnp.zeros(nr.shape, jnp.float32)
    for r in range(N_REL):
        logits = logits + jnp.where(nr == r, d[:, r:r + 1], 0.0)
    mx = jnp.max(logits, axis=1, keepdims=True)
    e = jnp.exp(logits - mx)
    att_ref[...] = e / jnp.sum(e, axis=1, keepdims=True)


def _tc_att(u_emb, rel, nr):
    blk = 256
    return pl.pallas_call(
        _att_body,
        grid=(B // blk,),
        in_specs=[pl.BlockSpec((blk, E_DIM), lambda i: (i, 0)),
                  pl.BlockSpec((N_REL, E_DIM), lambda i: (0, 0)),
                  pl.BlockSpec((blk, N_NEI), lambda i: (i, 0))],
        out_specs=pl.BlockSpec((blk, N_NEI), lambda i: (i, 0)),
        out_shape=jax.ShapeDtypeStruct((B, N_NEI), jnp.float32),
    )(u_emb, rel, nr)


def _final_body(u1_ref, ue_ref, it_ref, agg_ref, w_ref, b_ref, out_ref):
    u1 = u1_ref[...]
    ue = ue_ref[...]
    it = it_ref[...]
    i2 = jnp.dot(it + agg_ref[...], w_ref[...],
                 preferred_element_type=jnp.float32,
                 precision=lax.Precision.HIGHEST) + b_ref[...]
    i2 = jnp.maximum(i2, 0.0)

    def rdot(x, y):
        return jnp.sum(x * y, axis=1, keepdims=True)

    def sig(x):
        return 1.0 / (1.0 + jnp.exp(-x))

    out = sig(rdot(u1, i2))
    out1 = sig(rdot(ue, it))
    out2 = sig(rdot(u1, it))
    out3 = sig(rdot(ue, i2))
    result = (out1 + out2 + out3) / 3.0
    mx = jnp.max(result)
    mn = jnp.min(result)
    result = 0.5 + (result - mn) / (mx - mn + 1e-5)
    out_ref[...] = jnp.tanh(out * result)


def _tc_final(u1, u_emb, item_emb, agg, w, b2d):
    return pl.pallas_call(
        _final_body,
        out_shape=jax.ShapeDtypeStruct((B, 1), jnp.float32),
    )(u1, u_emb, item_emb, agg, w, b2d)


def kernel(user_table, entity_table, relation_table, E0, W1, b1, W2, b2,
           W_kgcn, b_kgcn, user_ids, item_ids, adj_entity, adj_relation,
           edge_index):
    src = edge_index[0].astype(jnp.int32)
    dst = edge_index[1].astype(jnp.int32)
    user_ids = user_ids.astype(jnp.int32)
    item_ids = item_ids.astype(jnp.int32)
    adj_entity = adj_entity.astype(jnp.int32)
    adj_relation = adj_relation.astype(jnp.int32)

    cat = jnp.concatenate([src, dst])
    cat = jnp.pad(cat, (0, DEG_PAD - 2 * N_EDGES), constant_values=N_NODES)
    cat = cat.reshape(NW, DEG_CHUNKS, CHUNK)
    ones_col = jnp.zeros((CHUNK, 16), jnp.float32).at[:, 0].set(1.0)
    (deg16,) = _deg_call(cat, ones_col)
    deg2d = deg16[:, :1]

    src_p = jnp.pad(src, (0, SEG_PAD - N_EDGES)).reshape(NS, SEG_CHUNKS, CHUNK)
    dst_p = jnp.pad(dst, (0, SEG_PAD - N_EDGES),
                    constant_values=N_NODES).reshape(NS, SEG_CHUNKS, CHUNK)

    b1r = b1.reshape(N_LAYERS, 1, E_DIM)
    b2r = b2.reshape(N_LAYERS, 1, E_DIM)

    ego = E0
    acc = E0
    egos = _tc_pre(E0, deg2d)
    for l in range(N_LAYERS):
        (raw,) = _seg_call(egos, src_p, dst_p)
        ego, acc, egos = _tc_layer(ego, acc, raw, deg2d,
                                   W1[l], b1r[l], W2[l], b2r[l])

    u_emb, item_emb, nr = _gath_call(user_table, entity_table, adj_relation,
                                     user_ids, item_ids)
    att = _tc_att(u_emb, relation_table, nr)
    (agg,) = _agg_call(entity_table, adj_entity, item_ids, att)
    (u1,) = _u1_call(acc, user_ids)

    out = _tc_final(u1, u_emb, item_emb, agg, W_kgcn,
                    b_kgcn.reshape(1, E_DIM))
    return out.reshape(B)


# speedup vs baseline: 2.6883x; 2.6883x over previous
import functools

import jax
import jax.numpy as jnp
from jax import lax
from jax.experimental import pallas as pl
from jax.experimental.pallas import tpu as pltpu
from jax.experimental.pallas import tpu_sc as plsc

N_USERS = 10000
N_ENT = 10000
N_REL = 16
E_DIM = 128
N_NEI = 32
B = 4096
N_EDGES = 320000
N_NODES = N_USERS + N_ENT
N_LAYERS = 3

NC = 2
NS = 16
NW = NC * NS
HALF = N_NODES // NC
ACC_ROWS = 10240
NP = NC * ACC_ROWS
HOLE = ACC_ROWS - HALF
CHUNK = 128

SEG_CHUNKS = 160
SEG_PER_W = SEG_CHUNKS * CHUNK
SEG_PAD = SEG_PER_W * NS

DEG_CHUNKS = 320
DEG_SLAB = 160
DEG_PER_S = DEG_CHUNKS * CHUNK
DEG_PAD = DEG_PER_S * NS

B_PER_W = B // NW

_mesh = plsc.VectorSubcoreMesh(core_axis_name="c", subcore_axis_name="s")


def _zero_spmem(shared, zbuf, s):
    rows_total = shared.shape[0]
    width = shared.shape[1]
    zrows = zbuf.shape[0]
    per_s = rows_total // NS

    def zf(i, _):
        for j in range(width // 16):
            zbuf[i, pl.ds(j * 16, 16)] = jnp.zeros((16,), jnp.float32)
        return 0
    lax.fori_loop(0, zrows, zf, 0)

    base = s * per_s
    off = 0
    left = per_s
    while left > 0:
        n = min(zrows, left)
        pltpu.sync_copy(zbuf.at[pl.ds(0, n)], shared.at[pl.ds(base + off, n)])
        off += n
        left -= n


def _deg_body(idx_hbm, ones_hbm, out_hbm, idx_v, ones_v, acc_sh, sem):
    c = lax.axis_index("c")
    s = lax.axis_index("s")
    w = c * NS + s
    _zero_spmem(acc_sh, ones_v, s)
    pltpu.sync_copy(ones_hbm, ones_v)
    plsc.subcore_barrier()

    def slab(t, _):
        pltpu.sync_copy(idx_hbm.at[c * NS + s, pl.ds(t * DEG_SLAB, DEG_SLAB)],
                        idx_v)

        def chunk(g, _):
            pltpu.sync_copy(ones_v, acc_sh.at[idx_v.at[g]], add=True)
            return 0
        lax.fori_loop(0, DEG_SLAB, chunk, 0)
        return 0
    lax.fori_loop(0, DEG_CHUNKS // DEG_SLAB, slab, 0)
    plsc.subcore_barrier()

    per_s = ACC_ROWS // NS
    row0 = s * per_s
    obase = c * ACC_ROWS
    for k in range(per_s // 128):
        off = row0 + k * 128
        pltpu.sync_copy(acc_sh.at[pl.ds(off, 128)], ones_v)
        pltpu.sync_copy(ones_v, out_hbm.at[pl.ds(obase + off, 128)])


_deg_call = functools.partial(
    pl.kernel,
    out_type=[jax.ShapeDtypeStruct((NP, E_DIM), jnp.float32)],
    mesh=_mesh,
    scratch_types=[
        pltpu.VMEM((DEG_SLAB, CHUNK), jnp.int32),
        pltpu.VMEM((CHUNK, E_DIM), jnp.float32),
        pltpu.VMEM_SHARED((ACC_ROWS, E_DIM), jnp.float32),
        pltpu.SemaphoreType.DMA,
    ],
)(_deg_body)


SLAB = 32
N_SLABS = SEG_CHUNKS // SLAB


def _seg_body(table_hbm, src_hbm, dst_hbm, out_hbm,
              src_v, dst_v, rows0, rows1, acc_sh, sem0, sem1):
    c = lax.axis_index("c")
    s = lax.axis_index("s")
    _zero_spmem(acc_sh, rows0, s)

    def slab(t, _):
        pltpu.sync_copy(src_hbm.at[s, pl.ds(t * SLAB, SLAB)], src_v)
        pltpu.sync_copy(dst_hbm.at[c * NS + s, pl.ds(t * SLAB, SLAB)], dst_v)

        def start(g, buf, sem):
            pltpu.async_copy(table_hbm.at[src_v.at[g]], buf, sem)

        def fin(g, buf, sem):
            pltpu.make_async_copy(table_hbm.at[src_v.at[g]], buf, sem).wait()
            pltpu.sync_copy(buf, acc_sh.at[dst_v.at[g]], add=True)

        start(0, rows0, sem0)
        start(1, rows1, sem1)

        def pair(p, _):
            g0 = p * 2
            fin(g0, rows0, sem0)
            start(g0 + 2, rows0, sem0)
            fin(g0 + 1, rows1, sem1)
            start(g0 + 3, rows1, sem1)
            return 0
        lax.fori_loop(0, SLAB // 2 - 1, pair, 0)
        fin(SLAB - 2, rows0, sem0)
        fin(SLAB - 1, rows1, sem1)
        return 0
    lax.fori_loop(0, N_SLABS, slab, 0)
    plsc.subcore_barrier()

    per_s = ACC_ROWS // NS
    row0 = s * per_s
    obase = c * ACC_ROWS
    for k in range(per_s // 128):
        off = row0 + k * 128
        pltpu.sync_copy(acc_sh.at[pl.ds(off, 128)], rows0.at[pl.ds(0, 128)])
        pltpu.sync_copy(rows0.at[pl.ds(0, 128)],
                        out_hbm.at[pl.ds(obase + off, 128)])


_seg_call = functools.partial(
    pl.kernel,
    out_type=[jax.ShapeDtypeStruct((NP, E_DIM), jnp.float32)],
    mesh=_mesh,
    scratch_types=[
        pltpu.VMEM((SLAB, CHUNK), jnp.int32),
        pltpu.VMEM((SLAB, CHUNK), jnp.int32),
        pltpu.VMEM((CHUNK, E_DIM), jnp.float32),
        pltpu.VMEM((CHUNK, E_DIM), jnp.float32),
        pltpu.VMEM_SHARED((ACC_ROWS, E_DIM), jnp.float32),
        pltpu.SemaphoreType.DMA,
        pltpu.SemaphoreType.DMA,
    ],
)(_seg_body)


def _gath_body(user_table, entity_table, adj_relation, user_ids, item_ids,
               u_out, it_out, nr_out, idx_v, rows_v, small_v, sem):
    c = lax.axis_index("c")
    s = lax.axis_index("s")
    w = c * NS + s
    b0 = w * B_PER_W
    pltpu.sync_copy(user_ids.at[pl.ds(b0, B_PER_W)], idx_v)
    pltpu.async_copy(user_table.at[idx_v], rows_v, sem).wait()
    pltpu.sync_copy(rows_v, u_out.at[pl.ds(b0, B_PER_W)])
    pltpu.sync_copy(item_ids.at[pl.ds(b0, B_PER_W)], idx_v)
    pltpu.async_copy(entity_table.at[idx_v], rows_v, sem).wait()
    pltpu.sync_copy(rows_v, it_out.at[pl.ds(b0, B_PER_W)])
    pltpu.async_copy(adj_relation.at[idx_v], small_v, sem).wait()
    pltpu.sync_copy(small_v, nr_out.at[pl.ds(b0, B_PER_W)])


_gath_call = functools.partial(
    pl.kernel,
    out_type=[
        jax.ShapeDtypeStruct((B, E_DIM), jnp.float32),
        jax.ShapeDtypeStruct((B, E_DIM), jnp.float32),
        jax.ShapeDtypeStruct((B, 128), jnp.int32),
    ],
    mesh=_mesh,
    scratch_types=[
        pltpu.VMEM((B_PER_W,), jnp.int32),
        pltpu.VMEM((B_PER_W, E_DIM), jnp.float32),
        pltpu.VMEM((B_PER_W, 128), jnp.int32),
        pltpu.SemaphoreType.DMA,
    ],
)(_gath_body)


def _agg_body(entity_table, adj_entity, item_ids, att_hbm, agg_out,
              idx_v, ne_v, att_v, agg_v, rows0, rows1, sem0, sem1):
    c = lax.axis_index("c")
    s = lax.axis_index("s")
    w = c * NS + s
    b0 = w * B_PER_W
    pltpu.sync_copy(item_ids.at[pl.ds(b0, B_PER_W)], idx_v)
    pltpu.async_copy(adj_entity.at[idx_v], ne_v, sem0).wait()
    pltpu.sync_copy(att_hbm.at[pl.ds(b0, B_PER_W)], att_v)

    def start(i, buf, sem):
        pltpu.async_copy(entity_table.at[ne_v.at[i, pl.ds(0, N_NEI)]], buf, sem)

    def item(i, buf):
        accs = [jnp.zeros((16,), jnp.float32) for _ in range(E_DIM // 16)]
        agrp = [att_v[i, pl.ds(g * 16, 16)] for g in range(N_NEI // 16)]
        for n in range(N_NEI):
            bn = agrp[n // 16].at[jnp.full((16,), n % 16, jnp.int32)].get(
                mode="promise_in_bounds")
            for v in range(E_DIM // 16):
                accs[v] = accs[v] + bn * buf[n, pl.ds(v * 16, 16)]
        for v in range(E_DIM // 16):
            agg_v[i, pl.ds(v * 16, 16)] = accs[v]

    def fin(i, buf, sem):
        pltpu.make_async_copy(
            entity_table.at[ne_v.at[i, pl.ds(0, N_NEI)]], buf, sem).wait()
        item(i, buf)

    start(0, rows0, sem0)
    start(1, rows1, sem1)

    def pair(p, _):
        i0 = p * 2
        fin(i0, rows0, sem0)
        start(i0 + 2, rows0, sem0)
        fin(i0 + 1, rows1, sem1)
        start(i0 + 3, rows1, sem1)
        return 0
    lax.fori_loop(0, B_PER_W // 2 - 1, pair, 0)
    fin(B_PER_W - 2, rows0, sem0)
    fin(B_PER_W - 1, rows1, sem1)
    pltpu.sync_copy(agg_v, agg_out.at[pl.ds(b0, B_PER_W)])


_agg_call = functools.partial(
    pl.kernel,
    out_type=[jax.ShapeDtypeStruct((B, E_DIM), jnp.float32)],
    mesh=_mesh,
    scratch_types=[
        pltpu.VMEM((B_PER_W,), jnp.int32),
        pltpu.VMEM((B_PER_W, 128), jnp.int32),
        pltpu.VMEM((B_PER_W, 128), jnp.float32),
        pltpu.VMEM((B_PER_W, E_DIM), jnp.float32),
        pltpu.VMEM((N_NEI, E_DIM), jnp.float32),
        pltpu.VMEM((N_NEI, E_DIM), jnp.float32),
        pltpu.SemaphoreType.DMA,
        pltpu.SemaphoreType.DMA,
    ],
)(_agg_body)


def _u1_body(acc_table, user_ids, u1_out, idx_v, rows_v, sem):
    c = lax.axis_index("c")
    s = lax.axis_index("s")
    w = c * NS + s
    b0 = w * B_PER_W
    pltpu.sync_copy(user_ids.at[pl.ds(b0, B_PER_W)], idx_v)
    pltpu.async_copy(acc_table.at[idx_v], rows_v, sem).wait()
    pltpu.sync_copy(rows_v, u1_out.at[pl.ds(b0, B_PER_W)])


_u1_call = functools.partial(
    pl.kernel,
    out_type=[jax.ShapeDtypeStruct((B, E_DIM), jnp.float32)],
    mesh=_mesh,
    scratch_types=[
        pltpu.VMEM((B_PER_W,), jnp.int32),
        pltpu.VMEM((B_PER_W, E_DIM), jnp.float32),
        pltpu.SemaphoreType.DMA,
    ],
)(_u1_body)


_NBLK = NP // 128


def _clamp_body(idx_ref, o0_ref, o1_ref):
    v = idx_ref[...]
    for c, o in ((0, o0_ref), (1, o1_ref)):
        d = v - c * HALF
        ok = (d >= 0) & (d < HALF)
        o[...] = jnp.where(ok, d, HALF)


def _tc_clamp(idx2d):
    rows = idx2d.shape[0]
    return pl.pallas_call(
        _clamp_body,
        grid=(rows // 128,),
        in_specs=[pl.BlockSpec((128, CHUNK), lambda i: (i, 0))],
        out_specs=[pl.BlockSpec((128, CHUNK), lambda i: (i, 0)),
                   pl.BlockSpec((128, CHUNK), lambda i: (i, 0))],
        out_shape=[jax.ShapeDtypeStruct((rows, CHUNK), jnp.int32),
                   jax.ShapeDtypeStruct((rows, CHUNK), jnp.int32)],
    )(idx2d)


def _srcmap_body(idx_ref, o_ref):
    v = idx_ref[...]
    o_ref[...] = jnp.where(v >= HALF, v + HOLE, v)


def _tc_srcmap(idx2d):
    rows = idx2d.shape[0]
    return pl.pallas_call(
        _srcmap_body,
        grid=(rows // 128,),
        in_specs=[pl.BlockSpec((128, CHUNK), lambda i: (i, 0))],
        out_specs=pl.BlockSpec((128, CHUNK), lambda i: (i, 0)),
        out_shape=jax.ShapeDtypeStruct((rows, CHUNK), jnp.int32),
    )(idx2d)


def _pre_body(e0_ref, deg_ref, egos_ref):
    rs = lax.rsqrt(jnp.maximum(deg_ref[...], 1.0))
    egos_ref[...] = e0_ref[...] * rs


def _tc_pre(E0, deg2d):
    return pl.pallas_call(
        _pre_body,
        grid=(_NBLK,),
        in_specs=[pl.BlockSpec((128, E_DIM), lambda i: (i, 0)),
                  pl.BlockSpec((128, 1), lambda i: (i, 0))],
        out_specs=pl.BlockSpec((128, E_DIM), lambda i: (i, 0)),
        out_shape=jax.ShapeDtypeStruct((NP, E_DIM), jnp.float32),
    )(E0, deg2d)


def _layer_body(ego_ref, acc_ref, raw_ref, deg_ref, w1_ref, b1_ref, w2_ref,
                b2_ref, ego_o, acc_o, egos_o):
    rs = lax.rsqrt(jnp.maximum(deg_ref[...], 1.0))
    ego = ego_ref[...]
    side = raw_ref[...] * rs
    a = side + ego
    m = side * ego
    h = (jnp.dot(a, w1_ref[...], preferred_element_type=jnp.float32,
                 precision=lax.Precision.HIGHEST)
         + jnp.dot(m, w2_ref[...], preferred_element_type=jnp.float32,
                   precision=lax.Precision.HIGHEST)
         + b1_ref[...] + b2_ref[...])
    h = jnp.where(h >= 0, h, 0.2 * h)
    nrm = jnp.sqrt(jnp.sum(h * h, axis=1, keepdims=True))
    hn = h / (nrm + 1e-12)
    ego_o[...] = hn
    acc_o[...] = acc_ref[...] + hn
    egos_o[...] = hn * rs


def _tc_layer(ego, acc, raw, deg2d, w1, b1, w2, b2):
    return pl.pallas_call(
        _layer_body,
        grid=(_NBLK,),
        in_specs=[pl.BlockSpec((128, E_DIM), lambda i: (i, 0)),
                  pl.BlockSpec((128, E_DIM), lambda i: (i, 0)),
                  pl.BlockSpec((128, E_DIM), lambda i: (i, 0)),
                  pl.BlockSpec((128, 1), lambda i: (i, 0)),
                  pl.BlockSpec((E_DIM, E_DIM), lambda i: (0, 0)),
                  pl.BlockSpec((1, E_DIM), lambda i: (0, 0)),
                  pl.BlockSpec((E_DIM, E_DIM), lambda i: (0, 0)),
                  pl.BlockSpec((1, E_DIM), lambda i: (0, 0))],
        out_specs=[pl.BlockSpec((128, E_DIM), lambda i: (i, 0)),
                   pl.BlockSpec((128, E_DIM), lambda i: (i, 0)),
                   pl.BlockSpec((128, E_DIM), lambda i: (i, 0))],
        out_shape=[jax.ShapeDtypeStruct((NP, E_DIM), jnp.float32),
                   jax.ShapeDtypeStruct((NP, E_DIM), jnp.float32),
                   jax.ShapeDtypeStruct((NP, E_DIM), jnp.float32)],
    )(ego, acc, raw, deg2d, w1, b1, w2, b2)


def _att_body(u_ref, rel_ref, nr_ref, att_ref):
    d = lax.dot_general(u_ref[...], rel_ref[...],
                        (((1,), (1,)), ((), ())),
                        preferred_element_type=jnp.float32,
                        precision=lax.Precision.HIGHEST)
    nr = nr_ref[...]
    logits = jnp.zeros(nr.shape, jnp.float32)
    for r in range(N_REL):
        logits = logits + jnp.where(nr == r, d[:, r:r + 1], 0.0)
    col = lax.broadcasted_iota(jnp.int32, nr.shape, 1)
    logits = jnp.where(col < N_NEI, logits, -1e30)
    mx = jnp.max(logits, axis=1, keepdims=True)
    e = jnp.exp(logits - mx)
    att_ref[...] = e / jnp.sum(e, axis=1, keepdims=True)


def _tc_att(u_emb, rel, nr):
    blk = 256
    return pl.pallas_call(
        _att_body,
        grid=(B // blk,),
        in_specs=[pl.BlockSpec((blk, E_DIM), lambda i: (i, 0)),
                  pl.BlockSpec((N_REL, E_DIM), lambda i: (0, 0)),
                  pl.BlockSpec((blk, 128), lambda i: (i, 0))],
        out_specs=pl.BlockSpec((blk, 128), lambda i: (i, 0)),
        out_shape=jax.ShapeDtypeStruct((B, 128), jnp.float32),
    )(u_emb, rel, nr)


def _final_body(u1_ref, ue_ref, it_ref, agg_ref, w_ref, b_ref, out_ref):
    u1 = u1_ref[...]
    ue = ue_ref[...]
    it = it_ref[...]
    i2 = jnp.dot(it + agg_ref[...], w_ref[...],
                 preferred_element_type=jnp.float32,
                 precision=lax.Precision.HIGHEST) + b_ref[...]
    i2 = jnp.maximum(i2, 0.0)

    def rdot(x, y):
        return jnp.sum(x * y, axis=1, keepdims=True)

    def sig(x):
        return 1.0 / (1.0 + jnp.exp(-x))

    out = sig(rdot(u1, i2))
    out1 = sig(rdot(ue, it))
    out2 = sig(rdot(u1, it))
    out3 = sig(rdot(ue, i2))
    result = (out1 + out2 + out3) / 3.0
    mx = jnp.max(result)
    mn = jnp.min(result)
    result = 0.5 + (result - mn) / (mx - mn + 1e-5)
    out_ref[...] = jnp.tanh(out * result)


def _tc_final(u1, u_emb, item_emb, agg, w, b2d):
    return pl.pallas_call(
        _final_body,
        out_shape=jax.ShapeDtypeStruct((B, 1), jnp.float32),
    )(u1, u_emb, item_emb, agg, w, b2d)


def kernel(user_table, entity_table, relation_table, E0, W1, b1, W2, b2,
           W_kgcn, b_kgcn, user_ids, item_ids, adj_entity, adj_relation,
           edge_index):
    src = edge_index[0].astype(jnp.int32)
    dst = edge_index[1].astype(jnp.int32)
    user_ids = user_ids.astype(jnp.int32)
    item_ids = item_ids.astype(jnp.int32)
    adj_entity = jnp.pad(adj_entity.astype(jnp.int32), ((0, 0), (0, 128 - N_NEI)))
    adj_relation = jnp.pad(adj_relation.astype(jnp.int32), ((0, 0), (0, 128 - N_NEI)))

    cat = jnp.concatenate([src, dst])
    cat = jnp.pad(cat, (0, DEG_PAD - 2 * N_EDGES), constant_values=N_NODES)
    ones_col = jnp.concatenate(
        [jnp.ones((CHUNK, 1), jnp.float32),
         jnp.zeros((CHUNK, E_DIM - 1), jnp.float32)], axis=1)
    c0, c1 = _tc_clamp(cat.reshape(NS * DEG_CHUNKS, CHUNK))
    catfix = jnp.stack([c0, c1]).reshape(NC * NS, DEG_CHUNKS, CHUNK)
    (deg16,) = _deg_call(catfix, ones_col)
    deg2d = deg16[:, :1]

    src_flat = jnp.pad(src, (0, SEG_PAD - N_EDGES))
    dst_flat = jnp.pad(dst, (0, SEG_PAD - N_EDGES), constant_values=N_NODES)
    src_p = _tc_srcmap(src_flat.reshape(NS * SEG_CHUNKS, CHUNK)).reshape(
        NS, SEG_CHUNKS, CHUNK)
    d0, d1 = _tc_clamp(dst_flat.reshape(NS * SEG_CHUNKS, CHUNK))
    dst_p = jnp.stack([d0, d1]).reshape(NC * NS, SEG_CHUNKS, CHUNK)

    b1r = b1.reshape(N_LAYERS, 1, E_DIM)
    b2r = b2.reshape(N_LAYERS, 1, E_DIM)

    zpad = jnp.zeros((HOLE, E_DIM), jnp.float32)
    E0p = jnp.concatenate([E0[:HALF], zpad, E0[HALF:], zpad])
    ego = E0p
    acc = E0p
    egos = _tc_pre(E0p, deg2d)
    for l in range(N_LAYERS):
        (raw,) = _seg_call(egos, src_p, dst_p)
        ego, acc, egos = _tc_layer(ego, acc, raw, deg2d,
                                   W1[l], b1r[l], W2[l], b2r[l])

    u_emb, item_emb, nr = _gath_call(user_table, entity_table,
                                     adj_relation, user_ids, item_ids)
    att = _tc_att(u_emb, relation_table, nr)
    (agg,) = _agg_call(entity_table, adj_entity, item_ids, att)
    (u1,) = _u1_call(acc, user_ids)

    out = _tc_final(u1, u_emb, item_emb, agg, W_kgcn,
                    b_kgcn.reshape(1, E_DIM))
    return out.reshape(B)
